# baseline (speedup 1.0000x reference)
import functools

import jax
import jax.numpy as jnp
from jax import lax
from jax.experimental import pallas as pl
from jax.experimental.pallas import tpu as pltpu

B, H, L, D, C = 2, 16, 4096, 64, 512
COMMITMENT = 0.0001
LBLK = 1024


def _fused_kernel(x_ref, m_ref, dists_ref, part_ref):
    xb = x_ref[0, 0]
    m = m_ref[0]
    n2 = jnp.sum(xb * xb, axis=-1, keepdims=True)
    xn = xb / jnp.maximum(jnp.sqrt(n2), 1e-12)
    d = jax.lax.dot_general(
        xn, m, (((1,), (1,)), ((), ())),
        preferred_element_type=jnp.float32)
    dists_ref[0, 0] = d
    maxv = jnp.max(d, axis=-1, keepdims=True)
    m2 = jnp.sum(m * m, axis=-1)
    m2row = jnp.max(
        jnp.where(d == maxv, m2[None, :], -jnp.inf), axis=-1)
    xn2 = jnp.sum(xn * xn, axis=-1)
    partial = (jnp.sum(xn2) - 2.0 * jnp.sum(maxv)
               + jnp.sum(m2row)).reshape(1, 1)

    @pl.when(pl.program_id(1) == 0)
    def _init():
        part_ref[0] = jnp.zeros((1, 1), jnp.float32)

    part_ref[0] += partial


@jax.jit
def kernel(x, means):
    BH = B * H
    grid = (BH, L // LBLK)
    dists, partials = pl.pallas_call(
        _fused_kernel,
        grid=grid,
        in_specs=[
            pl.BlockSpec((1, 1, LBLK, D), lambda i, j: (i // H, i % H, j, 0)),
            pl.BlockSpec((1, C, D), lambda i, j: (i % H, 0, 0)),
        ],
        out_specs=[
            pl.BlockSpec((1, 1, LBLK, C), lambda i, j: (i // H, i % H, j, 0)),
            pl.BlockSpec((1, 1, 1), lambda i, j: (i, 0, 0)),
        ],
        out_shape=[
            jax.ShapeDtypeStruct((B, H, L, C), jnp.float32),
            jax.ShapeDtypeStruct((BH, 1, 1), jnp.float32),
        ],
        compiler_params=pltpu.CompilerParams(
            dimension_semantics=("parallel", "arbitrary")),
    )(x, means)
    loss = jnp.sum(partials) * (COMMITMENT / (B * H * L * D))
    return dists, loss


# speedup vs baseline: 1.1696x; 1.1696x over previous
import functools

import jax
import jax.numpy as jnp
from jax import lax
from jax.experimental import pallas as pl
from jax.experimental.pallas import tpu as pltpu

B, H, L, D, C = 2, 16, 4096, 64, 512
COMMITMENT = 0.0001
LBLK = 2048


def _fused_kernel(x_ref, m_ref, dists_ref, part_ref):
    xb = x_ref[0, 0]
    m = m_ref[0]
    n2 = jnp.sum(xb * xb, axis=-1, keepdims=True)
    xn = xb / jnp.maximum(jnp.sqrt(n2), 1e-12)
    d = jax.lax.dot_general(
        xn, m, (((1,), (1,)), ((), ())),
        preferred_element_type=jnp.float32)
    dists_ref[0, 0] = d
    maxv = jnp.max(d, axis=-1, keepdims=True)
    m2 = jnp.sum(m * m, axis=-1)
    m2row = jnp.max(
        jnp.where(d == maxv, m2[None, :], -jnp.inf), axis=-1)
    xn2 = jnp.sum(xn * xn, axis=-1)
    partial = (jnp.sum(xn2) - 2.0 * jnp.sum(maxv)
               + jnp.sum(m2row)).reshape(1, 1)

    @pl.when(pl.program_id(1) == 0)
    def _init():
        part_ref[0] = jnp.zeros((1, 1), jnp.float32)

    part_ref[0] += partial


@jax.jit
def kernel(x, means):
    BH = B * H
    grid = (BH, L // LBLK)
    dists, partials = pl.pallas_call(
        _fused_kernel,
        grid=grid,
        in_specs=[
            pl.BlockSpec((1, 1, LBLK, D), lambda i, j: (i // H, i % H, j, 0)),
            pl.BlockSpec((1, C, D), lambda i, j: (i % H, 0, 0)),
        ],
        out_specs=[
            pl.BlockSpec((1, 1, LBLK, C), lambda i, j: (i // H, i % H, j, 0)),
            pl.BlockSpec((1, 1, 1), lambda i, j: (i, 0, 0)),
        ],
        out_shape=[
            jax.ShapeDtypeStruct((B, H, L, C), jnp.float32),
            jax.ShapeDtypeStruct((BH, 1, 1), jnp.float32),
        ],
        compiler_params=pltpu.CompilerParams(
            dimension_semantics=("parallel", "arbitrary")),
    )(x, means)
    loss = jnp.sum(partials) * (COMMITMENT / (B * H * L * D))
    return dists, loss


# speedup vs baseline: 1.3825x; 1.1821x over previous
import functools

import jax
import jax.numpy as jnp
from jax import lax
from jax.experimental import pallas as pl
from jax.experimental.pallas import tpu as pltpu

B, H, L, D, C = 2, 16, 4096, 64, 512
COMMITMENT = 0.0001
LBLK = 4096


def _fused_kernel(x_ref, m_ref, dists_ref, part_ref):
    xb = x_ref[0, 0]
    m = m_ref[0]
    n2 = jnp.sum(xb * xb, axis=-1, keepdims=True)
    xn = xb / jnp.maximum(jnp.sqrt(n2), 1e-12)
    d = jax.lax.dot_general(
        xn, m, (((1,), (1,)), ((), ())),
        preferred_element_type=jnp.float32)
    dists_ref[0, 0] = d
    maxv = jnp.max(d, axis=-1, keepdims=True)
    m2 = jnp.sum(m * m, axis=-1)
    m2row = jnp.max(
        jnp.where(d == maxv, m2[None, :], -jnp.inf), axis=-1)
    xn2 = jnp.sum(xn * xn, axis=-1)
    partial = (jnp.sum(xn2) - 2.0 * jnp.sum(maxv)
               + jnp.sum(m2row)).reshape(1, 1)

    @pl.when(pl.program_id(1) == 0)
    def _init():
        part_ref[0] = jnp.zeros((1, 1), jnp.float32)

    part_ref[0] += partial


@jax.jit
def kernel(x, means):
    BH = B * H
    grid = (BH, L // LBLK)
    dists, partials = pl.pallas_call(
        _fused_kernel,
        grid=grid,
        in_specs=[
            pl.BlockSpec((1, 1, LBLK, D), lambda i, j: (i // H, i % H, j, 0)),
            pl.BlockSpec((1, C, D), lambda i, j: (i % H, 0, 0)),
        ],
        out_specs=[
            pl.BlockSpec((1, 1, LBLK, C), lambda i, j: (i // H, i % H, j, 0)),
            pl.BlockSpec((1, 1, 1), lambda i, j: (i, 0, 0)),
        ],
        out_shape=[
            jax.ShapeDtypeStruct((B, H, L, C), jnp.float32),
            jax.ShapeDtypeStruct((BH, 1, 1), jnp.float32),
        ],
        compiler_params=pltpu.CompilerParams(
            dimension_semantics=("parallel", "arbitrary")),
    )(x, means)
    loss = jnp.sum(partials) * (COMMITMENT / (B * H * L * D))
    return dists, loss
